# Initial kernel scaffold; baseline (speedup 1.0000x reference)
import jax
import jax.numpy as jnp
from jax.experimental import pallas as pl


def kernel(slice_probs):
    raise NotImplementedError("write your pallas kernel here")


# speedup vs baseline: 16.7220x; 16.7220x over previous
import jax
import jax.numpy as jnp
from jax.experimental import pallas as pl
from jax.experimental.pallas import tpu as pltpu

_MAX_BIN = 16
_RB = 1024
_UNROLL = 8


def _dp_kernel(x_ref, o_ref):
    t_total = x_ref.shape[1]
    zeros = jnp.zeros((8, 128), jnp.float32)
    ones = jnp.ones((8, 128), jnp.float32)
    init = (ones,) + (zeros,) * _MAX_BIN

    def body(i, dp):
        ps = x_ref[0, pl.ds(i * _UNROLL, _UNROLL)]
        for j in range(_UNROLL):
            p = ps[j]
            q = 1.0 - p
            new = [dp[0] * q]
            for k in range(1, _MAX_BIN + 1):
                new.append(dp[k] * q + dp[k - 1] * p)
            new[_MAX_BIN] = new[_MAX_BIN] + dp[_MAX_BIN]
            dp = tuple(new)
        return dp

    dp = jax.lax.fori_loop(0, t_total // _UNROLL, body, init)
    for k in range(_MAX_BIN + 1):
        o_ref[0, k] = dp[k]


def kernel(slice_probs) -> jnp.ndarray:
    B, T = slice_probs.shape
    nb = B // _RB
    xt = jnp.transpose(slice_probs.reshape(nb, 8, 128, T), (0, 3, 1, 2))
    out = pl.pallas_call(
        _dp_kernel,
        grid=(nb,),
        in_specs=[pl.BlockSpec((1, T, 8, 128), lambda i: (i, 0, 0, 0))],
        out_specs=pl.BlockSpec((1, _MAX_BIN + 1, 8, 128), lambda i: (i, 0, 0, 0)),
        out_shape=jax.ShapeDtypeStruct((nb, _MAX_BIN + 1, 8, 128), jnp.float32),
        compiler_params=pltpu.CompilerParams(
            dimension_semantics=("parallel",),
        ),
        name="soft_count_dp",
    )(xt)
    return out.transpose(0, 2, 3, 1).reshape(B, _MAX_BIN + 1)
